# baseline (speedup 1.0000x reference)
import functools

import jax
import jax.numpy as jnp
from jax import lax
from jax.experimental import pallas as pl
from jax.experimental.pallas import tpu as pltpu
from jax.experimental.pallas import tpu_sc as plsc

NC = 2
NS = 16
CH = 128
BUFS = 5


def _sc_accumulate(f0, f1, idx3d, zrows, two_n_pad, groups_per_tile):
    rows_per_tile = two_n_pad // NS
    ng = groups_per_tile
    mesh = plsc.VectorSubcoreMesh(core_axis_name="c", subcore_axis_name="s")

    @functools.partial(
        pl.kernel,
        out_type=jax.ShapeDtypeStruct((NC, two_n_pad, 64), jnp.float32),
        mesh=mesh,
        scratch_types=[
            pltpu.VMEM((2, 2 * BUFS, CH), jnp.int32),
            pltpu.VMEM((BUFS, CH, 64), jnp.float32),
            pltpu.VMEM_SHARED((two_n_pad, 64), jnp.float32),
            pltpu.SemaphoreType.DMA((BUFS,)),
            pltpu.SemaphoreType.DMA,
        ],
        compiler_params=pltpu.CompilerParams(use_tc_tiling_on_sc=False),
    )
    def k(f0_hbm, f1_hbm, idx_hbm, zer_hbm, out_hbm,
          idx, rows, acc, gsem, isem):
        c = lax.axis_index("c")
        s = lax.axis_index("s")

        pltpu.sync_copy(zer_hbm, acc.at[pl.ds(s * rows_per_tile, rows_per_tile)])
        plsc.subcore_barrier()

        base = s * ng

        def main(f_hbm):
            def g_fire(b, m):
                pltpu.async_copy(f_hbm.at[idx.at[m].at[b]], rows.at[b],
                                 gsem.at[b])

            def g_wait(b, m):
                pltpu.make_async_copy(f_hbm.at[idx.at[m].at[b]], rows.at[b],
                                      gsem.at[b]).wait()

            def s_sync(b, m):
                pltpu.sync_copy(rows.at[b], acc.at[idx.at[m].at[BUFS + b]],
                                add=True)

            def idx_fire(m, grp):
                return pltpu.async_copy(idx_hbm.at[grp], idx.at[m], isem)

            def idx_wait():
                pltpu.make_async_copy(idx_hbm.at[0], idx.at[0], isem).wait()

            idx_fire(0, base).wait()
            idx_fire(1, base + 1)
            for b in range(BUFS):
                g_fire(b, 0)

            def phase(g, m):
                @pl.when(g <= ng - 2)
                def _():
                    idx_wait()
                for b in range(BUFS):
                    g_wait(b, m)
                    s_sync(b, m)

                    @pl.when(g <= ng - 2)
                    def _():
                        g_fire(b, 1 - m)

                @pl.when(g <= ng - 3)
                def _():
                    idx_fire(m, base + g + 2)

            def body(g2, _):
                phase(2 * g2, 0)
                phase(2 * g2 + 1, 1)
                return 0

            lax.fori_loop(0, ng // 2, body, 0)

        @pl.when(c == 0)
        def _():
            main(f0_hbm)

        @pl.when(c == 1)
        def _():
            main(f1_hbm)

        plsc.subcore_barrier()
        pltpu.sync_copy(
            acc.at[pl.ds(s * rows_per_tile, rows_per_tile)],
            out_hbm.at[c, pl.ds(s * rows_per_tile, rows_per_tile)],
        )

    return k(f0, f1, idx3d, zrows)


def _tc_tail_body(acc_ref, lin_ref, mw_ref, mb_ref, g_ref, b_ref, out_ref,
                  *, n_nodes, bn_eps):
    n = n_nodes
    h = jnp.zeros((n, 128), dtype=jnp.float32)
    for k in range(2):
        for c in range(2):
            a = acc_ref[c, k * n:(k + 1) * n, :]
            w = lin_ref[k, c * 64:(c + 1) * 64, :]
            h = h + jnp.dot(a, w, preferred_element_type=jnp.float32)
    z = jnp.dot(h, mw_ref[...].T, preferred_element_type=jnp.float32) + mb_ref[...]
    r = jnp.maximum(z, 0.0)
    mean = jnp.mean(r, axis=0, keepdims=True)
    var = jnp.mean((r - mean) * (r - mean), axis=0, keepdims=True)
    out_ref[...] = g_ref[...] * (r - mean) * lax.rsqrt(var + bn_eps) + b_ref[...]


def kernel(feature, sp_embeddings, edge_index, edge_order, linear, mlp_w,
           mlp_b, bn_gamma, bn_beta):
    n_nodes, in_feat = feature.shape
    e = edge_index.shape[1]
    assert in_feat == 128

    two_n_pad = ((2 * n_nodes + NS * 8) // (NS * 8)) * (NS * 8)

    grp_edges = NS * CH * BUFS * 2
    e_pad = ((e + grp_edges - 1) // grp_edges) * grp_edges
    src = edge_index[0]
    cidx = edge_index[1] + edge_order * n_nodes
    if e_pad != e:
        pad = e_pad - e
        src = jnp.concatenate([src, jnp.zeros((pad,), jnp.int32)])
        cidx = jnp.concatenate(
            [cidx, jnp.full((pad,), 2 * n_nodes, jnp.int32)])
    groups_per_tile = e_pad // (NS * CH * BUFS)
    nblk = NS * groups_per_tile
    idx3d = jnp.concatenate(
        [src.reshape(nblk, BUFS, CH), cidx.reshape(nblk, BUFS, CH)],
        axis=1)
    f0 = feature[:, :64]
    f1 = feature[:, 64:]
    zrows = jnp.zeros((two_n_pad // NS, 64), dtype=jnp.float32)

    acc = _sc_accumulate(f0, f1, idx3d, zrows, two_n_pad, groups_per_tile)

    tail = pl.pallas_call(
        functools.partial(_tc_tail_body, n_nodes=n_nodes, bn_eps=1e-5),
        out_shape=jax.ShapeDtypeStruct((n_nodes, 128), jnp.float32),
    )
    return tail(acc, linear, mlp_w, mlp_b.reshape(1, 128),
                bn_gamma.reshape(1, 128), bn_beta.reshape(1, 128))
